# Initial kernel scaffold; baseline (speedup 1.0000x reference)
import jax
import jax.numpy as jnp
from jax.experimental import pallas as pl


def kernel(ssl_content, proj_w, proj_b, codebook):
    raise NotImplementedError("write your pallas kernel here")


# speedup vs baseline: 1.3848x; 1.3848x over previous
import functools

import jax
import jax.numpy as jnp
from jax.experimental import pallas as pl
from jax.experimental.pallas import tpu as pltpu

B, C, T, K = 8, 768, 2048, 1024
TBLK = 512


def _cnorm_kernel(cb_ref, cnorm_ref):
    cb = cb_ref[...]
    cnorm_ref[...] = jnp.sum(cb * cb, axis=1, keepdims=True)


def _codes_kernel(w_ref, pb_ref, cb_ref, cnorm_ref, ssl_ref, out_ref):
    s = ssl_ref[0]
    x = jnp.dot(w_ref[...], s, preferred_element_type=jnp.float32) + pb_ref[...]
    dots = jnp.dot(cb_ref[...], x, preferred_element_type=jnp.float32)
    znorm = jnp.sum(x * x, axis=0, keepdims=True)
    d = (znorm - 2.0 * dots) + cnorm_ref[...]
    out_ref[0, 0, :] = jnp.argmin(d, axis=0).astype(jnp.int32)


@functools.partial(jax.jit, static_argnames=())
def kernel(ssl_content, proj_w, proj_b, codebook):
    cnorm = pl.pallas_call(
        _cnorm_kernel,
        out_shape=jax.ShapeDtypeStruct((K, 1), jnp.float32),
    )(codebook)

    codes = pl.pallas_call(
        _codes_kernel,
        grid=(B, T // TBLK),
        in_specs=[
            pl.BlockSpec((C, C), lambda b, t: (0, 0)),
            pl.BlockSpec((C, 1), lambda b, t: (0, 0)),
            pl.BlockSpec((K, C), lambda b, t: (0, 0)),
            pl.BlockSpec((K, 1), lambda b, t: (0, 0)),
            pl.BlockSpec((1, C, TBLK), lambda b, t: (b, 0, t)),
        ],
        out_specs=pl.BlockSpec((1, 1, TBLK), lambda b, t: (b, 0, t)),
        out_shape=jax.ShapeDtypeStruct((B, 1, T), jnp.int32),
        compiler_params=pltpu.CompilerParams(
            dimension_semantics=("parallel", "parallel")),
    )(proj_w, proj_b.reshape(C, 1), codebook, cnorm, ssl_content)

    return codes.reshape(B, T)
